# Initial kernel scaffold; baseline (speedup 1.0000x reference)
import jax
import jax.numpy as jnp
from jax.experimental import pallas as pl


def kernel(x, edge_index, W1l, b1l, W1r, W2l, b2l, W2r):
    raise NotImplementedError("write your pallas kernel here")


# speedup vs baseline: 10.2323x; 10.2323x over previous
import functools

import jax
import jax.numpy as jnp
from jax import lax
from jax.experimental import pallas as pl
from jax.experimental.pallas import tpu as pltpu
from jax.experimental.pallas import tpu_sc as plsc

N = 10000
D_IN = 128
H1 = 32
H2 = 64

NC = 2
NS = 16
NW = NC * NS
CH = 128
N_PAD = 10240
BN = 2000


def _mm_split(x, WlT, WrT):
    def body(x_r, wl_r, wr_r, p_r, r_r):
        xb = x_r[...]
        p_r[...] = jnp.dot(xb, wl_r[...], preferred_element_type=jnp.float32)
        r_r[...] = jnp.dot(xb, wr_r[...], preferred_element_type=jnp.float32)
    grid = N // BN
    return pl.pallas_call(
        body,
        grid=(grid,),
        in_specs=[
            pl.BlockSpec((BN, D_IN), lambda i: (i, 0)),
            pl.BlockSpec((D_IN, H1), lambda i: (0, 0)),
            pl.BlockSpec((D_IN, H1), lambda i: (0, 0)),
        ],
        out_specs=[
            pl.BlockSpec((BN, H1), lambda i: (i, 0)),
            pl.BlockSpec((BN, H1), lambda i: (i, 0)),
        ],
        out_shape=[
            jax.ShapeDtypeStruct((N, H1), jnp.float32),
            jax.ShapeDtypeStruct((N, H1), jnp.float32),
        ],
    )(x, WlT, WrT)


def _layer1_elem(agg, deg, R1, b1):
    def body(a_r, d_r, r_r, b_r, h_r):
        a = a_r[...]
        d = d_r[...]
        s = a[0] + a[1]
        dd = jnp.maximum(d[0] + d[1], 1.0)
        h_r[...] = jnp.maximum(s / dd + b_r[...] + r_r[...], 0.0)
    grid = N // BN
    return pl.pallas_call(
        body,
        grid=(grid,),
        in_specs=[
            pl.BlockSpec((NC, BN, H1), lambda i: (0, i, 0)),
            pl.BlockSpec((NC, BN, H1), lambda i: (0, i, 0)),
            pl.BlockSpec((BN, H1), lambda i: (i, 0)),
            pl.BlockSpec((1, H1), lambda i: (0, 0)),
        ],
        out_specs=pl.BlockSpec((BN, H1), lambda i: (i, 0)),
        out_shape=jax.ShapeDtypeStruct((N, H1), jnp.float32),
    )(agg, deg, R1, b1)


def _layer2_out(agg, deg, h, WlT, WrT, b2):
    def body(a_r, d_r, h_r, wl_r, wr_r, b_r, o_r):
        a = a_r[...]
        d = d_r[...]
        m = (a[0] + a[1]) / jnp.maximum(d[0] + d[1], 1.0)
        o_r[...] = (
            jnp.dot(m, wl_r[...], preferred_element_type=jnp.float32)
            + jnp.dot(h_r[...], wr_r[...], preferred_element_type=jnp.float32)
            + b_r[...]
        )
    grid = N // BN
    return pl.pallas_call(
        body,
        grid=(grid,),
        in_specs=[
            pl.BlockSpec((NC, BN, H1), lambda i: (0, i, 0)),
            pl.BlockSpec((NC, BN, H1), lambda i: (0, i, 0)),
            pl.BlockSpec((BN, H1), lambda i: (i, 0)),
            pl.BlockSpec((H1, H2), lambda i: (0, 0)),
            pl.BlockSpec((H1, H2), lambda i: (0, 0)),
            pl.BlockSpec((1, H2), lambda i: (0, 0)),
        ],
        out_specs=pl.BlockSpec((BN, H2), lambda i: (i, 0)),
        out_shape=jax.ShapeDtypeStruct((N, H2), jnp.float32),
    )(agg, deg, h, WlT, WrT, b2)


def _sc_segsum(table, src_i, dst_i, zeros, ones, with_deg):
    nch = src_i.shape[1]
    mesh = plsc.VectorSubcoreMesh(core_axis_name="c", subcore_axis_name="s")
    n_acc = 2 if with_deg else 1
    out_type = [jax.ShapeDtypeStruct((NC, N_PAD, H1), jnp.float32)] * n_acc
    scratch = [
        pltpu.VMEM((nch, CH), jnp.int32),
        pltpu.VMEM((nch, CH), jnp.int32),
        pltpu.VMEM((CH, H1), jnp.float32),
        pltpu.VMEM((CH, H1), jnp.float32),
        pltpu.SemaphoreType.DMA,
    ] + [pltpu.VMEM_SHARED((N_PAD, H1), jnp.float32)] * n_acc

    rpt = N_PAD // NS

    @functools.partial(pl.kernel, mesh=mesh, out_type=out_type,
                       scratch_types=scratch,
                       compiler_params=pltpu.CompilerParams(
                           use_tc_tiling_on_sc=False))
    def k(table_h, src_h, dst_h, zeros_h, ones_h, *rest):
        outs = rest[:n_acc]
        src_v, dst_v, rows_v, ones_v, sem = rest[n_acc:n_acc + 5]
        accs = rest[n_acc + 5:]
        cid = lax.axis_index("c")
        sid = lax.axis_index("s")
        wid = sid * NC + cid
        pltpu.sync_copy(src_h.at[wid], src_v)
        pltpu.sync_copy(dst_h.at[wid], dst_v)
        if with_deg:
            pltpu.sync_copy(ones_h, ones_v)
        row_sl = pl.ds(sid * rpt, rpt)
        for acc in accs:
            pltpu.sync_copy(zeros_h.at[row_sl], acc.at[row_sl])
        plsc.subcore_barrier()

        def step(j, carry):
            pltpu.async_copy(table_h.at[src_v.at[j]], rows_v, sem).wait()
            pltpu.sync_copy(rows_v, accs[0].at[dst_v.at[j]], add=True)
            if with_deg:
                pltpu.sync_copy(ones_v, accs[1].at[dst_v.at[j]], add=True)
            return carry

        lax.fori_loop(0, nch, step, 0)
        plsc.subcore_barrier()
        for acc, out in zip(accs, outs):
            pltpu.sync_copy(acc.at[row_sl], out.at[cid, row_sl])

    res = k(table, src_i, dst_i, zeros, ones)
    return res if with_deg else res[0] if isinstance(res, (tuple, list)) else res


def kernel(x, edge_index, W1l, b1l, W1r, W2l, b2l, W2r):
    E = edge_index.shape[1]
    nch = -(-E // (NW * CH))
    e_pad = NW * nch * CH
    pad = e_pad - E
    src = jnp.concatenate(
        [edge_index[0], jnp.zeros((pad,), jnp.int32)]).reshape(NW, nch, CH)
    dst = jnp.concatenate(
        [edge_index[1], jnp.full((pad,), N, jnp.int32)]).reshape(NW, nch, CH)
    zeros = jnp.zeros((N_PAD, H1), jnp.float32)
    ones = jnp.ones((CH, H1), jnp.float32)

    P1, R1 = _mm_split(x, W1l.T, W1r.T)
    agg1, deg = _sc_segsum(P1, src, dst, zeros, ones, with_deg=True)
    h = _layer1_elem(agg1, deg, R1, b1l.reshape(1, H1))
    agg2 = _sc_segsum(h, src, dst, zeros, ones, with_deg=False)
    out = _layer2_out(agg2, deg, h, W2l.T, W2r.T, b2l.reshape(1, H2))
    return out


# speedup vs baseline: 14.3683x; 1.4042x over previous
import functools

import jax
import jax.numpy as jnp
from jax import lax
from jax.experimental import pallas as pl
from jax.experimental.pallas import tpu as pltpu
from jax.experimental.pallas import tpu_sc as plsc

N = 10000
D_IN = 128
H1 = 32
H2 = 64
DW = 16

NC = 2
NS = 16
NW = NC * NS
CH = 128
NB = 8
N_PAD = 10240
BN = 2000


def _mm_split(x, WlT, WrT):
    def body(x_r, wl_r, wr_r, p_r, r_r):
        xb = x_r[...]
        p_r[...] = jnp.dot(xb, wl_r[...], preferred_element_type=jnp.float32)
        r_r[...] = jnp.dot(xb, wr_r[...], preferred_element_type=jnp.float32)
    grid = N // BN
    return pl.pallas_call(
        body,
        grid=(grid,),
        in_specs=[
            pl.BlockSpec((BN, D_IN), lambda i: (i, 0)),
            pl.BlockSpec((D_IN, H1), lambda i: (0, 0)),
            pl.BlockSpec((D_IN, H1), lambda i: (0, 0)),
        ],
        out_specs=[
            pl.BlockSpec((BN, H1), lambda i: (i, 0)),
            pl.BlockSpec((BN, H1), lambda i: (i, 0)),
        ],
        out_shape=[
            jax.ShapeDtypeStruct((N, H1), jnp.float32),
            jax.ShapeDtypeStruct((N, H1), jnp.float32),
        ],
    )(x, WlT, WrT)


def _layer1_elem(agg, deg, R1, b1):
    def body(a_r, d_r, r_r, b_r, h_r):
        a = a_r[...]
        d = d_r[...]
        s = a[0] + a[1]
        dd = jnp.maximum(d[0][:, :1] + d[1][:, :1], 1.0)
        h_r[...] = jnp.maximum(s / dd + b_r[...] + r_r[...], 0.0)
    grid = N // BN
    return pl.pallas_call(
        body,
        grid=(grid,),
        in_specs=[
            pl.BlockSpec((NC, BN, H1), lambda i: (0, i, 0)),
            pl.BlockSpec((NC, BN, DW), lambda i: (0, i, 0)),
            pl.BlockSpec((BN, H1), lambda i: (i, 0)),
            pl.BlockSpec((1, H1), lambda i: (0, 0)),
        ],
        out_specs=pl.BlockSpec((BN, H1), lambda i: (i, 0)),
        out_shape=jax.ShapeDtypeStruct((N, H1), jnp.float32),
    )(agg, deg, R1, b1)


def _layer2_out(agg, deg, h, WlT, WrT, b2):
    def body(a_r, d_r, h_r, wl_r, wr_r, b_r, o_r):
        a = a_r[...]
        d = d_r[...]
        m = (a[0] + a[1]) / jnp.maximum(d[0][:, :1] + d[1][:, :1], 1.0)
        o_r[...] = (
            jnp.dot(m, wl_r[...], preferred_element_type=jnp.float32)
            + jnp.dot(h_r[...], wr_r[...], preferred_element_type=jnp.float32)
            + b_r[...]
        )
    grid = N // BN
    return pl.pallas_call(
        body,
        grid=(grid,),
        in_specs=[
            pl.BlockSpec((NC, BN, H1), lambda i: (0, i, 0)),
            pl.BlockSpec((NC, BN, DW), lambda i: (0, i, 0)),
            pl.BlockSpec((BN, H1), lambda i: (i, 0)),
            pl.BlockSpec((H1, H2), lambda i: (0, 0)),
            pl.BlockSpec((H1, H2), lambda i: (0, 0)),
            pl.BlockSpec((1, H2), lambda i: (0, 0)),
        ],
        out_specs=pl.BlockSpec((BN, H2), lambda i: (i, 0)),
        out_shape=jax.ShapeDtypeStruct((N, H2), jnp.float32),
    )(agg, deg, h, WlT, WrT, b2)


def _sc_segsum(table, src_i, dst_i, z32, z16, ones, with_deg):
    nch = src_i.shape[1]
    mesh = plsc.VectorSubcoreMesh(core_axis_name="c", subcore_axis_name="s")
    out_type = [jax.ShapeDtypeStruct((NC, N_PAD, H1), jnp.float32)]
    scratch = [
        pltpu.VMEM((nch, CH), jnp.int32),
        pltpu.VMEM((nch, CH), jnp.int32),
        pltpu.VMEM((NB, CH, H1), jnp.float32),
        pltpu.VMEM((CH, DW), jnp.float32),
        pltpu.SemaphoreType.DMA((NB,)),
        pltpu.SemaphoreType.DMA((NB,)),
        pltpu.SemaphoreType.DMA((NB,)),
        pltpu.VMEM_SHARED((N_PAD, H1), jnp.float32),
    ]
    if with_deg:
        out_type.append(jax.ShapeDtypeStruct((NC, N_PAD, DW), jnp.float32))
        scratch.append(pltpu.VMEM_SHARED((N_PAD, DW), jnp.float32))

    rpt = N_PAD // NS

    @functools.partial(pl.kernel, mesh=mesh, out_type=out_type,
                       scratch_types=scratch,
                       compiler_params=pltpu.CompilerParams(
                           use_tc_tiling_on_sc=False))
    def k(table_h, src_h, dst_h, z32_h, z16_h, ones_h, *rest):
        if with_deg:
            (agg_o, deg_o, src_v, dst_v, rows_v, ones_v,
             gsem, ssem, dsem, acc_s, deg_s) = rest
        else:
            (agg_o, src_v, dst_v, rows_v, ones_v,
             gsem, ssem, dsem, acc_s) = rest
        cid = lax.axis_index("c")
        sid = lax.axis_index("s")
        wid = sid * NC + cid
        pltpu.sync_copy(src_h.at[wid], src_v)
        pltpu.sync_copy(dst_h.at[wid], dst_v)
        if with_deg:
            pltpu.sync_copy(ones_h, ones_v)
        row_sl = pl.ds(sid * rpt, rpt)
        pltpu.sync_copy(z32_h.at[row_sl], acc_s.at[row_sl])
        if with_deg:
            pltpu.sync_copy(z16_h.at[row_sl], deg_s.at[row_sl])
        plsc.subcore_barrier()

        for b in range(NB):
            pltpu.async_copy(table_h.at[src_v.at[b]], rows_v.at[b],
                             gsem.at[b])

        def step(i, carry):
            b = lax.rem(i, NB)
            pltpu.make_async_copy(table_h.at[src_v.at[i]], rows_v.at[b],
                                  gsem.at[b]).wait()
            sc = pltpu.async_copy(rows_v.at[b], acc_s.at[dst_v.at[i]],
                                  ssem.at[b], add=True)
            if with_deg:
                dc = pltpu.async_copy(ones_v, deg_s.at[dst_v.at[i]],
                                      dsem.at[b], add=True)
            sc.wait()
            if with_deg:
                dc.wait()
            nxt = i + NB
            @pl.when(nxt < nch)
            def _():
                pltpu.async_copy(table_h.at[src_v.at[nxt]], rows_v.at[b],
                                 gsem.at[b])
            return carry

        lax.fori_loop(0, nch, step, 0)
        plsc.subcore_barrier()
        pltpu.sync_copy(acc_s.at[row_sl], agg_o.at[cid, row_sl])
        if with_deg:
            pltpu.sync_copy(deg_s.at[row_sl], deg_o.at[cid, row_sl])

    res = k(table, src_i, dst_i, z32, z16, ones)
    return res if with_deg else res[0]


def kernel(x, edge_index, W1l, b1l, W1r, W2l, b2l, W2r):
    E = edge_index.shape[1]
    nch = -(-E // (NW * CH))
    e_pad = NW * nch * CH
    pad = e_pad - E
    src = jnp.concatenate(
        [edge_index[0], jnp.zeros((pad,), jnp.int32)]).reshape(NW, nch, CH)
    dst = jnp.concatenate(
        [edge_index[1], jnp.full((pad,), N, jnp.int32)]).reshape(NW, nch, CH)
    z32 = jnp.zeros((N_PAD, H1), jnp.float32)
    z16 = jnp.zeros((N_PAD, DW), jnp.float32)
    ones = jnp.ones((CH, DW), jnp.float32)

    P1, R1 = _mm_split(x, W1l.T, W1r.T)
    agg1, deg = _sc_segsum(P1, src, dst, z32, z16, ones, with_deg=True)
    h = _layer1_elem(agg1, deg, R1, b1l.reshape(1, H1))
    agg2 = _sc_segsum(h, src, dst, z32, z16, ones, with_deg=False)
    out = _layer2_out(agg2, deg, h, W2l.T, W2r.T, b2l.reshape(1, H2))
    return out


# speedup vs baseline: 18.6875x; 1.3006x over previous
import functools

import jax
import jax.numpy as jnp
from jax import lax
from jax.experimental import pallas as pl
from jax.experimental.pallas import tpu as pltpu
from jax.experimental.pallas import tpu_sc as plsc

N = 10000
D_IN = 128
H1 = 32
H2 = 64
DW = 16

NC = 2
NS = 16
NW = NC * NS
CH = 128
NB = 8
N_PAD = 10240
BN = 2000


def _mm_split(x, WlT, WrT):
    def body(x_r, wl_r, wr_r, p_r, r_r):
        xb = x_r[...]
        p_r[...] = jnp.dot(xb, wl_r[...], preferred_element_type=jnp.float32)
        r_r[...] = jnp.dot(xb, wr_r[...], preferred_element_type=jnp.float32)
    grid = N // BN
    return pl.pallas_call(
        body,
        grid=(grid,),
        in_specs=[
            pl.BlockSpec((BN, D_IN), lambda i: (i, 0)),
            pl.BlockSpec((D_IN, H1), lambda i: (0, 0)),
            pl.BlockSpec((D_IN, H1), lambda i: (0, 0)),
        ],
        out_specs=[
            pl.BlockSpec((BN, H1), lambda i: (i, 0)),
            pl.BlockSpec((BN, H1), lambda i: (i, 0)),
        ],
        out_shape=[
            jax.ShapeDtypeStruct((N, H1), jnp.float32),
            jax.ShapeDtypeStruct((N, H1), jnp.float32),
        ],
    )(x, WlT, WrT)


def _layer1_elem(agg, deg, R1, b1):
    def body(a_r, d_r, r_r, b_r, h_r):
        a = a_r[...]
        d = d_r[...]
        s = a[0] + a[1]
        dd = jnp.maximum(d[0][:, :1] + d[1][:, :1], 1.0)
        h_r[...] = jnp.maximum(s / dd + b_r[...] + r_r[...], 0.0)
    grid = N // BN
    return pl.pallas_call(
        body,
        grid=(grid,),
        in_specs=[
            pl.BlockSpec((NC, BN, H1), lambda i: (0, i, 0)),
            pl.BlockSpec((NC, BN, DW), lambda i: (0, i, 0)),
            pl.BlockSpec((BN, H1), lambda i: (i, 0)),
            pl.BlockSpec((1, H1), lambda i: (0, 0)),
        ],
        out_specs=pl.BlockSpec((BN, H1), lambda i: (i, 0)),
        out_shape=jax.ShapeDtypeStruct((N, H1), jnp.float32),
    )(agg, deg, R1, b1)


def _layer2_out(agg, deg, h, WlT, WrT, b2):
    def body(a_r, d_r, h_r, wl_r, wr_r, b_r, o_r):
        a = a_r[...]
        d = d_r[...]
        m = (a[0] + a[1]) / jnp.maximum(d[0][:, :1] + d[1][:, :1], 1.0)
        o_r[...] = (
            jnp.dot(m, wl_r[...], preferred_element_type=jnp.float32)
            + jnp.dot(h_r[...], wr_r[...], preferred_element_type=jnp.float32)
            + b_r[...]
        )
    grid = N // BN
    return pl.pallas_call(
        body,
        grid=(grid,),
        in_specs=[
            pl.BlockSpec((NC, BN, H1), lambda i: (0, i, 0)),
            pl.BlockSpec((NC, BN, DW), lambda i: (0, i, 0)),
            pl.BlockSpec((BN, H1), lambda i: (i, 0)),
            pl.BlockSpec((H1, H2), lambda i: (0, 0)),
            pl.BlockSpec((H1, H2), lambda i: (0, 0)),
            pl.BlockSpec((1, H2), lambda i: (0, 0)),
        ],
        out_specs=pl.BlockSpec((BN, H2), lambda i: (i, 0)),
        out_shape=jax.ShapeDtypeStruct((N, H2), jnp.float32),
    )(agg, deg, h, WlT, WrT, b2)


def _sc_segsum(table, src_i, dst_i, z32, z16, ones, with_deg):
    nch = src_i.shape[1]
    mesh = plsc.VectorSubcoreMesh(core_axis_name="c", subcore_axis_name="s")
    out_type = [jax.ShapeDtypeStruct((NC, N_PAD, H1), jnp.float32)]
    scratch = [
        pltpu.VMEM((nch, CH), jnp.int32),
        pltpu.VMEM((nch, CH), jnp.int32),
        pltpu.VMEM((NB, CH, H1), jnp.float32),
        pltpu.VMEM((CH, DW), jnp.float32),
        pltpu.SemaphoreType.DMA((NB,)),
        pltpu.SemaphoreType.DMA((NB,)),
        pltpu.SemaphoreType.DMA((NB,)),
        pltpu.VMEM_SHARED((N_PAD, H1), jnp.float32),
        pltpu.VMEM_SHARED((N, H1), jnp.float32),
    ]
    if with_deg:
        out_type.append(jax.ShapeDtypeStruct((NC, N_PAD, DW), jnp.float32))
        scratch.append(pltpu.VMEM_SHARED((N_PAD, DW), jnp.float32))

    rpt = N_PAD // NS

    @functools.partial(pl.kernel, mesh=mesh, out_type=out_type,
                       scratch_types=scratch,
                       compiler_params=pltpu.CompilerParams(
                           use_tc_tiling_on_sc=False))
    def k(table_h, src_h, dst_h, z32_h, z16_h, ones_h, *rest):
        if with_deg:
            (agg_o, deg_o, src_v, dst_v, rows_v, ones_v,
             gsem, ssem, dsem, acc_s, table_s, deg_s) = rest
        else:
            (agg_o, src_v, dst_v, rows_v, ones_v,
             gsem, ssem, dsem, acc_s, table_s) = rest
        cid = lax.axis_index("c")
        sid = lax.axis_index("s")
        wid = sid * NC + cid
        pltpu.sync_copy(src_h.at[wid], src_v)
        pltpu.sync_copy(dst_h.at[wid], dst_v)
        if with_deg:
            pltpu.sync_copy(ones_h, ones_v)
        row_sl = pl.ds(sid * rpt, rpt)
        pltpu.sync_copy(z32_h.at[row_sl], acc_s.at[row_sl])
        if with_deg:
            pltpu.sync_copy(z16_h.at[row_sl], deg_s.at[row_sl])
        tpt = N // NS
        tab_sl = pl.ds(sid * tpt, tpt)
        pltpu.sync_copy(table_h.at[tab_sl], table_s.at[tab_sl])
        plsc.subcore_barrier()

        for b in range(NB):
            pltpu.async_copy(table_s.at[src_v.at[b]], rows_v.at[b],
                             gsem.at[b])

        def step(i, carry):
            b = lax.rem(i, NB)
            pltpu.make_async_copy(table_s.at[src_v.at[i]], rows_v.at[b],
                                  gsem.at[b]).wait()
            sc = pltpu.async_copy(rows_v.at[b], acc_s.at[dst_v.at[i]],
                                  ssem.at[b], add=True)
            if with_deg:
                dc = pltpu.async_copy(ones_v, deg_s.at[dst_v.at[i]],
                                      dsem.at[b], add=True)
            sc.wait()
            if with_deg:
                dc.wait()
            nxt = i + NB
            @pl.when(nxt < nch)
            def _():
                pltpu.async_copy(table_s.at[src_v.at[nxt]], rows_v.at[b],
                                 gsem.at[b])
            return carry

        lax.fori_loop(0, nch, step, 0)
        plsc.subcore_barrier()
        pltpu.sync_copy(acc_s.at[row_sl], agg_o.at[cid, row_sl])
        if with_deg:
            pltpu.sync_copy(deg_s.at[row_sl], deg_o.at[cid, row_sl])

    res = k(table, src_i, dst_i, z32, z16, ones)
    return res if with_deg else res[0]


def kernel(x, edge_index, W1l, b1l, W1r, W2l, b2l, W2r):
    E = edge_index.shape[1]
    nch = -(-E // (NW * CH))
    e_pad = NW * nch * CH
    pad = e_pad - E
    src = jnp.concatenate(
        [edge_index[0], jnp.zeros((pad,), jnp.int32)]).reshape(NW, nch, CH)
    dst = jnp.concatenate(
        [edge_index[1], jnp.full((pad,), N, jnp.int32)]).reshape(NW, nch, CH)
    z32 = jnp.zeros((N_PAD, H1), jnp.float32)
    z16 = jnp.zeros((N_PAD, DW), jnp.float32)
    ones = jnp.ones((CH, DW), jnp.float32)

    P1, R1 = _mm_split(x, W1l.T, W1r.T)
    agg1, deg = _sc_segsum(P1, src, dst, z32, z16, ones, with_deg=True)
    h = _layer1_elem(agg1, deg, R1, b1l.reshape(1, H1))
    agg2 = _sc_segsum(h, src, dst, z32, z16, ones, with_deg=False)
    out = _layer2_out(agg2, deg, h, W2l.T, W2r.T, b2l.reshape(1, H2))
    return out


# speedup vs baseline: 23.7256x; 1.2696x over previous
import functools

import jax
import jax.numpy as jnp
from jax import lax
from jax.experimental import pallas as pl
from jax.experimental.pallas import tpu as pltpu
from jax.experimental.pallas import tpu_sc as plsc

N = 10000
D_IN = 128
H1 = 32
H2 = 64
PK = 4
NP = N // PK

NC = 2
NS = 16
NW = NC * NS
EPW_CH = 80
NCH = 125
NB = 8
N_PAD = N
NP_PAD = N_PAD // PK
BNP = NP


def _mm_packed(x4, bdl, bdr):
    def body(x_r, wl_r, wr_r, p_r, r_r):
        xb = x_r[...]
        p_r[...] = jnp.dot(xb, wl_r[...], preferred_element_type=jnp.float32)
        r_r[...] = jnp.dot(xb, wr_r[...], preferred_element_type=jnp.float32)
    grid = NP // BNP
    return pl.pallas_call(
        body,
        grid=(grid,),
        in_specs=[
            pl.BlockSpec((BNP, PK * D_IN), lambda i: (i, 0)),
            pl.BlockSpec((PK * D_IN, PK * H1), lambda i: (0, 0)),
            pl.BlockSpec((PK * D_IN, PK * H1), lambda i: (0, 0)),
        ],
        out_specs=[
            pl.BlockSpec((BNP, PK * H1), lambda i: (i, 0)),
            pl.BlockSpec((BNP, PK * H1), lambda i: (i, 0)),
        ],
        out_shape=[
            jax.ShapeDtypeStruct((NP, PK * H1), jnp.float32),
            jax.ShapeDtypeStruct((NP, PK * H1), jnp.float32),
        ],
    )(x4, bdl, bdr)


def _layer1_elem(agg, deg, R1p, b1p):
    def body(a_r, d_r, r_r, b_r, h_r):
        a = a_r[...]
        d = d_r[...]
        s = a[0] + a[1]
        dd = jnp.maximum(d[0] + d[1], 1.0)
        h_r[...] = jnp.maximum(s / dd + b_r[...] + r_r[...], 0.0)
    grid = NP // BNP
    return pl.pallas_call(
        body,
        grid=(grid,),
        in_specs=[
            pl.BlockSpec((NC, BNP, PK * H1), lambda i: (0, i, 0)),
            pl.BlockSpec((NC, BNP, PK * H1), lambda i: (0, i, 0)),
            pl.BlockSpec((BNP, PK * H1), lambda i: (i, 0)),
            pl.BlockSpec((1, PK * H1), lambda i: (0, 0)),
        ],
        out_specs=pl.BlockSpec((BNP, PK * H1), lambda i: (i, 0)),
        out_shape=jax.ShapeDtypeStruct((NP, PK * H1), jnp.float32),
    )(agg, deg, R1p, b1p)


def _layer2_out(agg, deg, hp, bd2, b2p):
    def body(a_r, d_r, h_r, w_r, b_r, o_r):
        a = a_r[...]
        d = d_r[...]
        m = (a[0] + a[1]) / jnp.maximum(d[0] + d[1], 1.0)
        mh = jnp.concatenate([m, h_r[...]], axis=1)
        o_r[...] = (
            jnp.dot(mh, w_r[...], preferred_element_type=jnp.float32)
            + b_r[...]
        )
    grid = NP // BNP
    return pl.pallas_call(
        body,
        grid=(grid,),
        in_specs=[
            pl.BlockSpec((NC, BNP, PK * H1), lambda i: (0, i, 0)),
            pl.BlockSpec((NC, BNP, PK * H1), lambda i: (0, i, 0)),
            pl.BlockSpec((BNP, PK * H1), lambda i: (i, 0)),
            pl.BlockSpec((2 * PK * H1, PK * H2), lambda i: (0, 0)),
            pl.BlockSpec((1, PK * H2), lambda i: (0, 0)),
        ],
        out_specs=pl.BlockSpec((BNP, PK * H2), lambda i: (i, 0)),
        out_shape=jax.ShapeDtypeStruct((NP, PK * H2), jnp.float32),
    )(agg, deg, hp, bd2, b2p)


def _sc_segsum(table, edges, zeros, ones, with_deg):
    epw = NCH * EPW_CH
    mesh = plsc.VectorSubcoreMesh(core_axis_name="c", subcore_axis_name="s")
    n_acc = 2 if with_deg else 1
    out_type = [jax.ShapeDtypeStruct((NC, N_PAD, H1), jnp.float32)] * n_acc
    scratch = [
        pltpu.VMEM((epw,), jnp.int32),
        pltpu.VMEM((NCH, EPW_CH), jnp.int32),
        pltpu.VMEM((NB, EPW_CH, H1), jnp.float32),
        pltpu.VMEM((EPW_CH, H1), jnp.float32),
        pltpu.SemaphoreType.DMA((NB,)),
        pltpu.SemaphoreType.DMA((NB,)),
        pltpu.SemaphoreType.DMA((NB,)),
        pltpu.SemaphoreType.DMA,
        pltpu.VMEM_SHARED((N, H1), jnp.float32),
        pltpu.VMEM_SHARED((N_PAD, H1), jnp.float32),
    ]
    if with_deg:
        scratch.append(pltpu.VMEM_SHARED((N_PAD, H1), jnp.float32))

    rpt = N_PAD // NS

    @functools.partial(pl.kernel, mesh=mesh, out_type=out_type,
                       scratch_types=scratch,
                       compiler_params=pltpu.CompilerParams(
                           use_tc_tiling_on_sc=False))
    def k(table_h, edge_h, zeros_h, ones_h, *rest):
        if with_deg:
            (agg_o, deg_o, src_v, dst_v, rows_v, ones_v,
             gsem, ssem, dsem, csem, table_s, acc_s, deg_s) = rest
        else:
            (agg_o, src_v, dst_v, rows_v, ones_v,
             gsem, ssem, dsem, csem, table_s, acc_s) = rest
        cid = lax.axis_index("c")
        sid = lax.axis_index("s")
        wid = sid * NC + cid
        base = wid * epw
        pltpu.sync_copy(edge_h.at[0, pl.ds(base, epw)], src_v)

        def dst_start(j, carry):
            pltpu.async_copy(edge_h.at[1, pl.ds(base + j * EPW_CH, EPW_CH)],
                             dst_v.at[j], csem)
            return carry

        lax.fori_loop(0, NCH, dst_start, 0)
        if with_deg:
            pltpu.sync_copy(ones_h, ones_v)
        row_sl = pl.ds(sid * rpt, rpt)
        pltpu.sync_copy(zeros_h.at[row_sl], acc_s.at[row_sl])
        if with_deg:
            pltpu.sync_copy(zeros_h.at[row_sl], deg_s.at[row_sl])
        tpt = N // NS
        tab_sl = pl.ds(sid * tpt, tpt)
        pltpu.sync_copy(table_h.at[tab_sl], table_s.at[tab_sl])

        def dst_wait(j, carry):
            pltpu.make_async_copy(
                edge_h.at[1, pl.ds(base + j * EPW_CH, EPW_CH)],
                dst_v.at[j], csem).wait()
            return carry

        lax.fori_loop(0, NCH, dst_wait, 0)
        plsc.subcore_barrier()

        for b in range(NB):
            pltpu.async_copy(
                table_s.at[src_v.at[pl.ds(b * EPW_CH, EPW_CH)]],
                rows_v.at[b], gsem.at[b])

        def step(i, carry):
            b = lax.rem(i, NB)
            src_sl = src_v.at[pl.ds(i * EPW_CH, EPW_CH)]
            pltpu.make_async_copy(table_s.at[src_sl], rows_v.at[b],
                                  gsem.at[b]).wait()
            sc = pltpu.async_copy(rows_v.at[b], acc_s.at[dst_v.at[i]],
                                  ssem.at[b], add=True)
            if with_deg:
                dc = pltpu.async_copy(ones_v, deg_s.at[dst_v.at[i]],
                                      dsem.at[b], add=True)
            sc.wait()
            if with_deg:
                dc.wait()
            nxt = i + NB
            @pl.when(nxt < NCH)
            def _():
                pltpu.async_copy(
                    table_s.at[src_v.at[pl.ds(nxt * EPW_CH, EPW_CH)]],
                    rows_v.at[b], gsem.at[b])
            return carry

        lax.fori_loop(0, NCH, step, 0)
        plsc.subcore_barrier()
        pltpu.sync_copy(acc_s.at[row_sl], agg_o.at[cid, row_sl])
        if with_deg:
            pltpu.sync_copy(deg_s.at[row_sl], deg_o.at[cid, row_sl])

    res = k(table, edges, zeros, ones)
    return res if with_deg else res[0]


def kernel(x, edge_index, W1l, b1l, W1r, W2l, b2l, W2r):
    eye = jnp.eye(PK, dtype=jnp.float32)
    bd1l = jnp.kron(eye, W1l.T)
    bd1r = jnp.kron(eye, W1r.T)
    bd2 = jnp.concatenate(
        [jnp.kron(eye, W2l.T), jnp.kron(eye, W2r.T)], axis=0)
    b1p = jnp.tile(b1l, PK).reshape(1, PK * H1)
    b2p = jnp.tile(b2l, PK).reshape(1, PK * H2)
    zeros = jnp.zeros((N_PAD, H1), jnp.float32)
    ones = jnp.ones((EPW_CH, H1), jnp.float32)
    x4 = x.reshape(NP, PK * D_IN)

    P1p, R1p = _mm_packed(x4, bd1l, bd1r)
    agg1, deg = _sc_segsum(P1p.reshape(N, H1), edge_index, zeros, ones,
                           with_deg=True)
    agg1p = agg1.reshape(NC, NP_PAD, PK * H1)
    degp = deg.reshape(NC, NP_PAD, PK * H1)
    hp = _layer1_elem(agg1p, degp, R1p, b1p)
    agg2 = _sc_segsum(hp.reshape(N, H1), edge_index, zeros, ones,
                      with_deg=False)
    agg2p = agg2.reshape(NC, NP_PAD, PK * H1)
    outp = _layer2_out(agg2p, degp, hp, bd2, b2p)
    return outp.reshape(N, H2)


# speedup vs baseline: 24.6057x; 1.0371x over previous
import functools

import jax
import jax.numpy as jnp
from jax import lax
from jax.experimental import pallas as pl
from jax.experimental.pallas import tpu as pltpu
from jax.experimental.pallas import tpu_sc as plsc

N = 10000
D_IN = 128
H1 = 32
H2 = 64
PK = 4
NP = N // PK

NC = 2
NS = 16
NW = NC * NS
EPW_CH = 80
NCH = 125
NB = 8
N_PAD = 10240
NP_PAD = N_PAD // PK


def _mm_packed(x4, bdl, bdr):
    def body(x_r, wl_r, wr_r, p_r, r_r):
        xb = x_r[...]
        p_r[...] = jnp.dot(xb, wl_r[...], preferred_element_type=jnp.float32)
        r_r[...] = jnp.dot(xb, wr_r[...], preferred_element_type=jnp.float32)
    BR = 640
    grid = -(-NP // BR)
    return pl.pallas_call(
        body,
        grid=(grid,),
        in_specs=[
            pl.BlockSpec((BR, PK * D_IN), lambda i: (i, 0)),
            pl.BlockSpec((PK * D_IN, PK * H1), lambda i: (0, 0)),
            pl.BlockSpec((PK * D_IN, PK * H1), lambda i: (0, 0)),
        ],
        out_specs=[
            pl.BlockSpec((BR, PK * H1), lambda i: (i, 0)),
            pl.BlockSpec((BR, PK * H1), lambda i: (i, 0)),
        ],
        out_shape=[
            jax.ShapeDtypeStruct((NP, PK * H1), jnp.float32),
            jax.ShapeDtypeStruct((NP, PK * H1), jnp.float32),
        ],
    )(x4, bdl, bdr)


def _layer1_elem(agg, deg, R1p, b1p):
    def body(a_r, d_r, r_r, b_r, h_r):
        a = a_r[...]
        d = d_r[...]
        s = a[0] + a[1]
        dd = jnp.maximum(d[0] + d[1], 1.0)
        h_r[...] = jnp.maximum(s / dd + b_r[...] + r_r[...], 0.0)
    BR = 640
    grid = -(-NP // BR)
    return pl.pallas_call(
        body,
        grid=(grid,),
        in_specs=[
            pl.BlockSpec((NC, BR, PK * H1), lambda i: (0, i, 0)),
            pl.BlockSpec((NC, BR, PK * H1), lambda i: (0, i, 0)),
            pl.BlockSpec((BR, PK * H1), lambda i: (i, 0)),
            pl.BlockSpec((1, PK * H1), lambda i: (0, 0)),
        ],
        out_specs=pl.BlockSpec((BR, PK * H1), lambda i: (i, 0)),
        out_shape=jax.ShapeDtypeStruct((NP, PK * H1), jnp.float32),
    )(agg, deg, R1p, b1p)


def _layer2_out(agg, deg, hp, bd2, b2p):
    def body(a_r, d_r, h_r, w_r, b_r, o_r):
        a = a_r[...]
        d = d_r[...]
        m = (a[0] + a[1]) / jnp.maximum(d[0] + d[1], 1.0)
        mh = jnp.concatenate([m, h_r[...]], axis=1)
        o_r[...] = (
            jnp.dot(mh, w_r[...], preferred_element_type=jnp.float32)
            + b_r[...]
        )
    BR = 640
    grid = -(-NP // BR)
    return pl.pallas_call(
        body,
        grid=(grid,),
        in_specs=[
            pl.BlockSpec((NC, BR, PK * H1), lambda i: (0, i, 0)),
            pl.BlockSpec((NC, BR, PK * H1), lambda i: (0, i, 0)),
            pl.BlockSpec((BR, PK * H1), lambda i: (i, 0)),
            pl.BlockSpec((2 * PK * H1, PK * H2), lambda i: (0, 0)),
            pl.BlockSpec((1, PK * H2), lambda i: (0, 0)),
        ],
        out_specs=pl.BlockSpec((BR, PK * H2), lambda i: (i, 0)),
        out_shape=jax.ShapeDtypeStruct((NP, PK * H2), jnp.float32),
    )(agg, deg, hp, bd2, b2p)


def _sc_segsum(table, edges, zeros, ones, with_deg):
    epw = NCH * EPW_CH
    mesh = plsc.VectorSubcoreMesh(core_axis_name="c", subcore_axis_name="s")
    n_acc = 2 if with_deg else 1
    out_type = [jax.ShapeDtypeStruct((NC, N_PAD, H1), jnp.float32)] * n_acc
    scratch = [
        pltpu.VMEM((epw,), jnp.int32),
        pltpu.VMEM((NCH, EPW_CH), jnp.int32),
        pltpu.VMEM((NB, EPW_CH, H1), jnp.float32),
        pltpu.VMEM((EPW_CH, H1), jnp.float32),
        pltpu.SemaphoreType.DMA((NB,)),
        pltpu.SemaphoreType.DMA((NB,)),
        pltpu.SemaphoreType.DMA((NB,)),
        pltpu.SemaphoreType.DMA,
        pltpu.VMEM_SHARED((N, H1), jnp.float32),
        pltpu.VMEM_SHARED((N_PAD, H1), jnp.float32),
    ]
    if with_deg:
        scratch.append(pltpu.VMEM_SHARED((N_PAD, H1), jnp.float32))

    rpt = N_PAD // NS

    @functools.partial(pl.kernel, mesh=mesh, out_type=out_type,
                       scratch_types=scratch,
                       compiler_params=pltpu.CompilerParams(
                           use_tc_tiling_on_sc=False))
    def k(table_h, edge_h, zeros_h, ones_h, *rest):
        if with_deg:
            (agg_o, deg_o, src_v, dst_v, rows_v, ones_v,
             gsem, ssem, dsem, csem, table_s, acc_s, deg_s) = rest
        else:
            (agg_o, src_v, dst_v, rows_v, ones_v,
             gsem, ssem, dsem, csem, table_s, acc_s) = rest
        cid = lax.axis_index("c")
        sid = lax.axis_index("s")
        wid = sid * NC + cid
        base = wid * epw
        pltpu.sync_copy(edge_h.at[0, pl.ds(base, epw)], src_v)

        def dst_start(j, carry):
            pltpu.async_copy(edge_h.at[1, pl.ds(base + j * EPW_CH, EPW_CH)],
                             dst_v.at[j], csem)
            return carry

        lax.fori_loop(0, NCH, dst_start, 0)
        if with_deg:
            pltpu.sync_copy(ones_h, ones_v)
        row_sl = pl.ds(sid * rpt, rpt)
        pltpu.sync_copy(zeros_h.at[row_sl], acc_s.at[row_sl])
        if with_deg:
            pltpu.sync_copy(zeros_h.at[row_sl], deg_s.at[row_sl])
        tpt = N // NS
        tab_sl = pl.ds(sid * tpt, tpt)
        pltpu.sync_copy(table_h.at[tab_sl], table_s.at[tab_sl])

        def dst_wait(j, carry):
            pltpu.make_async_copy(
                edge_h.at[1, pl.ds(base + j * EPW_CH, EPW_CH)],
                dst_v.at[j], csem).wait()
            return carry

        lax.fori_loop(0, NCH, dst_wait, 0)
        plsc.subcore_barrier()

        for b in range(NB):
            pltpu.async_copy(
                table_s.at[src_v.at[pl.ds(b * EPW_CH, EPW_CH)]],
                rows_v.at[b], gsem.at[b])

        def step(i, carry):
            b = lax.rem(i, NB)
            src_sl = src_v.at[pl.ds(i * EPW_CH, EPW_CH)]
            pltpu.make_async_copy(table_s.at[src_sl], rows_v.at[b],
                                  gsem.at[b]).wait()
            sc = pltpu.async_copy(rows_v.at[b], acc_s.at[dst_v.at[i]],
                                  ssem.at[b], add=True)
            if with_deg:
                @pl.when(i >= NB)
                def _():
                    pltpu.make_async_copy(ones_v, deg_s.at[dst_v.at[i]],
                                          dsem.at[b]).wait()
                pltpu.async_copy(ones_v, deg_s.at[dst_v.at[i]],
                                 dsem.at[b], add=True)
            sc.wait()
            nxt = i + NB
            @pl.when(nxt < NCH)
            def _():
                pltpu.async_copy(
                    table_s.at[src_v.at[pl.ds(nxt * EPW_CH, EPW_CH)]],
                    rows_v.at[b], gsem.at[b])
            return carry

        lax.fori_loop(0, NCH, step, 0)
        if with_deg:
            def deg_drain(j, carry):
                pltpu.make_async_copy(ones_v, deg_s.at[dst_v.at[j]],
                                      dsem.at[lax.rem(j, NB)]).wait()
                return carry
            lax.fori_loop(NCH - NB, NCH, deg_drain, 0)
        plsc.subcore_barrier()
        pltpu.sync_copy(acc_s.at[row_sl], agg_o.at[cid, row_sl])
        if with_deg:
            pltpu.sync_copy(deg_s.at[row_sl], deg_o.at[cid, row_sl])

    res = k(table, edges, zeros, ones)
    return res if with_deg else res[0]


def kernel(x, edge_index, W1l, b1l, W1r, W2l, b2l, W2r):
    eye = jnp.eye(PK, dtype=jnp.float32)
    bd1l = jnp.kron(eye, W1l.T)
    bd1r = jnp.kron(eye, W1r.T)
    bd2 = jnp.concatenate(
        [jnp.kron(eye, W2l.T), jnp.kron(eye, W2r.T)], axis=0)
    b1p = jnp.tile(b1l, PK).reshape(1, PK * H1)
    b2p = jnp.tile(b2l, PK).reshape(1, PK * H2)
    zeros = jnp.zeros((N_PAD, H1), jnp.float32)
    ones = jnp.ones((EPW_CH, H1), jnp.float32)
    x4 = x.reshape(NP, PK * D_IN)

    P1p, R1p = _mm_packed(x4, bd1l, bd1r)
    agg1, deg = _sc_segsum(P1p.reshape(N, H1), edge_index, zeros, ones,
                           with_deg=True)
    agg1p = agg1.reshape(NC, NP_PAD, PK * H1)
    degp = deg.reshape(NC, NP_PAD, PK * H1)
    hp = _layer1_elem(agg1p, degp, R1p, b1p)
    agg2 = _sc_segsum(hp.reshape(N, H1), edge_index, zeros, ones,
                      with_deg=False)
    agg2p = agg2.reshape(NC, NP_PAD, PK * H1)
    outp = _layer2_out(agg2p, degp, hp, bd2, b2p)
    return outp.reshape(N, H2)


# speedup vs baseline: 24.9091x; 1.0123x over previous
import functools

import jax
import jax.numpy as jnp
from jax import lax
from jax.experimental import pallas as pl
from jax.experimental.pallas import tpu as pltpu
from jax.experimental.pallas import tpu_sc as plsc

N = 10000
D_IN = 128
H1 = 32
H2 = 64
PK = 4
NP = N // PK

NC = 2
NS = 16
NW = NC * NS
EPW_CH = 80
NCH = 125
NB = 8
N_PAD = 10240
NP_PAD = N_PAD // PK


def _mm_packed(x4, bdl, bdr):
    def body(x_r, wl_r, wr_r, p_r, r_r):
        xb = x_r[...]
        p_r[...] = jnp.dot(xb, wl_r[...], preferred_element_type=jnp.float32)
        r_r[...] = jnp.dot(xb, wr_r[...], preferred_element_type=jnp.float32)
    BR = 640
    grid = -(-NP // BR)
    return pl.pallas_call(
        body,
        grid=(grid,),
        in_specs=[
            pl.BlockSpec((BR, PK * D_IN), lambda i: (i, 0)),
            pl.BlockSpec((PK * D_IN, PK * H1), lambda i: (0, 0)),
            pl.BlockSpec((PK * D_IN, PK * H1), lambda i: (0, 0)),
        ],
        out_specs=[
            pl.BlockSpec((BR, PK * H1), lambda i: (i, 0)),
            pl.BlockSpec((BR, PK * H1), lambda i: (i, 0)),
        ],
        out_shape=[
            jax.ShapeDtypeStruct((NP, PK * H1), jnp.float32),
            jax.ShapeDtypeStruct((NP, PK * H1), jnp.float32),
        ],
    )(x4, bdl, bdr)


def _layer1_elem(agg, deg, R1p, b1p):
    def body(a_r, d_r, r_r, b_r, h_r):
        a = a_r[...]
        d = d_r[...]
        s = a[0] + a[1]
        dd = jnp.maximum(d[0] + d[1], 1.0)
        h_r[...] = jnp.maximum(s / dd + b_r[...] + r_r[...], 0.0)
    BR = 640
    grid = -(-NP // BR)
    return pl.pallas_call(
        body,
        grid=(grid,),
        in_specs=[
            pl.BlockSpec((NC, BR, PK * H1), lambda i: (0, i, 0)),
            pl.BlockSpec((NC, BR, PK * H1), lambda i: (0, i, 0)),
            pl.BlockSpec((BR, PK * H1), lambda i: (i, 0)),
            pl.BlockSpec((1, PK * H1), lambda i: (0, 0)),
        ],
        out_specs=pl.BlockSpec((BR, PK * H1), lambda i: (i, 0)),
        out_shape=jax.ShapeDtypeStruct((NP, PK * H1), jnp.float32),
    )(agg, deg, R1p, b1p)


def _layer2_out(agg, deg, hp, bd2, b2p):
    def body(a_r, d_r, h_r, w_r, b_r, o_r):
        a = a_r[...]
        d = d_r[...]
        m = (a[0] + a[1]) / jnp.maximum(d[0] + d[1], 1.0)
        mh = jnp.concatenate([m, h_r[...]], axis=1)
        o_r[...] = (
            jnp.dot(mh, w_r[...], preferred_element_type=jnp.float32)
            + b_r[...]
        )
    BR = 640
    grid = -(-NP // BR)
    return pl.pallas_call(
        body,
        grid=(grid,),
        in_specs=[
            pl.BlockSpec((NC, BR, PK * H1), lambda i: (0, i, 0)),
            pl.BlockSpec((NC, BR, PK * H1), lambda i: (0, i, 0)),
            pl.BlockSpec((BR, PK * H1), lambda i: (i, 0)),
            pl.BlockSpec((2 * PK * H1, PK * H2), lambda i: (0, 0)),
            pl.BlockSpec((1, PK * H2), lambda i: (0, 0)),
        ],
        out_specs=pl.BlockSpec((BR, PK * H2), lambda i: (i, 0)),
        out_shape=jax.ShapeDtypeStruct((NP, PK * H2), jnp.float32),
    )(agg, deg, hp, bd2, b2p)


def _sc_segsum(table, edges, zeros, ones, with_deg):
    epw = NCH * EPW_CH
    mesh = plsc.VectorSubcoreMesh(core_axis_name="c", subcore_axis_name="s")
    n_acc = 2 if with_deg else 1
    out_type = [jax.ShapeDtypeStruct((NC, N_PAD, H1), jnp.float32)] * n_acc
    scratch = [
        pltpu.VMEM((epw,), jnp.int32),
        pltpu.VMEM((NCH, EPW_CH), jnp.int32),
        pltpu.VMEM((NB, EPW_CH, H1), jnp.float32),
        pltpu.VMEM((EPW_CH, H1), jnp.float32),
        pltpu.SemaphoreType.DMA((NB,)),
        pltpu.SemaphoreType.DMA((NB,)),
        pltpu.SemaphoreType.DMA((NB,)),
        pltpu.SemaphoreType.DMA,
        pltpu.VMEM_SHARED((N, H1), jnp.float32),
        pltpu.VMEM_SHARED((N_PAD, H1), jnp.float32),
    ]
    if with_deg:
        scratch.append(pltpu.VMEM_SHARED((N_PAD, H1), jnp.float32))

    rpt = N_PAD // NS

    @functools.partial(pl.kernel, mesh=mesh, out_type=out_type,
                       scratch_types=scratch,
                       compiler_params=pltpu.CompilerParams(
                           use_tc_tiling_on_sc=False))
    def k(table_h, edge_h, zeros_h, ones_h, *rest):
        if with_deg:
            (agg_o, deg_o, src_v, dst_v, rows_v, ones_v,
             gsem, ssem, dsem, csem, table_s, acc_s, deg_s) = rest
        else:
            (agg_o, src_v, dst_v, rows_v, ones_v,
             gsem, ssem, dsem, csem, table_s, acc_s) = rest
        cid = lax.axis_index("c")
        sid = lax.axis_index("s")
        wid = sid * NC + cid
        base = wid * epw
        pltpu.sync_copy(edge_h.at[0, pl.ds(base, epw)], src_v)

        def dst_start(j, carry):
            pltpu.async_copy(edge_h.at[1, pl.ds(base + j * EPW_CH, EPW_CH)],
                             dst_v.at[j], csem)
            return carry

        lax.fori_loop(0, NCH, dst_start, 0)
        if with_deg:
            pltpu.sync_copy(ones_h, ones_v)
        row_sl = pl.ds(sid * rpt, rpt)
        pltpu.sync_copy(zeros_h.at[row_sl], acc_s.at[row_sl])
        if with_deg:
            pltpu.sync_copy(zeros_h.at[row_sl], deg_s.at[row_sl])
        tpt = N // NS
        tab_sl = pl.ds(sid * tpt, tpt)
        pltpu.sync_copy(table_h.at[tab_sl], table_s.at[tab_sl])

        def dst_wait(j, carry):
            pltpu.make_async_copy(
                edge_h.at[1, pl.ds(base + j * EPW_CH, EPW_CH)],
                dst_v.at[j], csem).wait()
            return carry

        lax.fori_loop(0, NCH, dst_wait, 0)
        plsc.subcore_barrier()

        for b in range(NB):
            pltpu.async_copy(
                table_s.at[src_v.at[pl.ds(b * EPW_CH, EPW_CH)]],
                rows_v.at[b], gsem.at[b])

        def step(i, carry):
            b = lax.rem(i, NB)
            src_sl = src_v.at[pl.ds(i * EPW_CH, EPW_CH)]
            pltpu.make_async_copy(table_s.at[src_sl], rows_v.at[b],
                                  gsem.at[b]).wait()
            sc = pltpu.async_copy(rows_v.at[b], acc_s.at[dst_v.at[i]],
                                  ssem.at[b], add=True)
            if with_deg:
                dc = pltpu.async_copy(ones_v, deg_s.at[dst_v.at[i]],
                                      dsem.at[b], add=True)
            sc.wait()
            if with_deg:
                dc.wait()
            nxt = i + NB
            @pl.when(nxt < NCH)
            def _():
                pltpu.async_copy(
                    table_s.at[src_v.at[pl.ds(nxt * EPW_CH, EPW_CH)]],
                    rows_v.at[b], gsem.at[b])
            return carry

        lax.fori_loop(0, NCH, step, 0)
        plsc.subcore_barrier()
        pltpu.sync_copy(acc_s.at[row_sl], agg_o.at[cid, row_sl])
        if with_deg:
            pltpu.sync_copy(deg_s.at[row_sl], deg_o.at[cid, row_sl])

    res = k(table, edges, zeros, ones)
    return res if with_deg else res[0]


def kernel(x, edge_index, W1l, b1l, W1r, W2l, b2l, W2r):
    eye = jnp.eye(PK, dtype=jnp.float32)
    bd1l = jnp.kron(eye, W1l.T)
    bd1r = jnp.kron(eye, W1r.T)
    bd2 = jnp.concatenate(
        [jnp.kron(eye, W2l.T), jnp.kron(eye, W2r.T)], axis=0)
    b1p = jnp.tile(b1l, PK).reshape(1, PK * H1)
    b2p = jnp.tile(b2l, PK).reshape(1, PK * H2)
    zeros = jnp.zeros((N_PAD, H1), jnp.float32)
    ones = jnp.ones((EPW_CH, H1), jnp.float32)
    x4 = x.reshape(NP, PK * D_IN)

    P1p, R1p = _mm_packed(x4, bd1l, bd1r)
    agg1, deg = _sc_segsum(P1p.reshape(N, H1), edge_index, zeros, ones,
                           with_deg=True)
    agg1p = agg1.reshape(NC, NP_PAD, PK * H1)
    degp = deg.reshape(NC, NP_PAD, PK * H1)
    hp = _layer1_elem(agg1p, degp, R1p, b1p)
    agg2 = _sc_segsum(hp.reshape(N, H1), edge_index, zeros, ones,
                      with_deg=False)
    agg2p = agg2.reshape(NC, NP_PAD, PK * H1)
    outp = _layer2_out(agg2p, degp, hp, bd2, b2p)
    return outp.reshape(N, H2)
